# Initial kernel scaffold; baseline (speedup 1.0000x reference)
import functools

import jax
import jax.numpy as jnp
from jax import lax
from jax.experimental import pallas as pl
from jax.experimental.pallas import tpu as pltpu
from jax.experimental.pallas import tpu_sc as plsc


@functools.partial(jax.jit, static_argnames=("n", "d"))
def _sc_gather(table, idx, n, d):
    info = plsc.get_sparse_core_info()
    nc, ns = info.num_cores, info.num_subcores
    nw = nc * ns
    assert n % nw == 0
    b_per_w = n // nw

    chunk = b_per_w
    while chunk > 1024 or chunk % 8 != 0:
        chunk //= 2
    n_chunks = b_per_w // chunk

    mesh = plsc.VectorSubcoreMesh(core_axis_name="c", subcore_axis_name="s")

    @functools.partial(
        pl.kernel,
        mesh=mesh,
        out_type=jax.ShapeDtypeStruct((n, d), jnp.float32),
        scratch_types=[
            pltpu.VMEM((chunk,), jnp.int32),
            pltpu.VMEM((chunk, d), jnp.float32),
            pltpu.SemaphoreType.DMA,
        ],
    )
    def k(table_hbm, idx_hbm, out_hbm, idx_v, rows_v, sem):
        wid = lax.axis_index("s") * nc + lax.axis_index("c")
        base = wid * b_per_w

        def step(g, carry):
            off = base + g * chunk
            pltpu.sync_copy(idx_hbm.at[pl.ds(off, chunk)], idx_v)
            pltpu.async_copy(table_hbm.at[idx_v], rows_v, sem).wait()
            pltpu.sync_copy(rows_v, out_hbm.at[pl.ds(off, chunk)])
            return carry

        lax.fori_loop(0, n_chunks, step, 0)

    return k(table, idx)


def kernel(text_vec, w2v_table):
    b, l = text_vec.shape
    v, d = w2v_table.shape
    n = b * l
    idx = text_vec.reshape(n).astype(jnp.int32)
    out = _sc_gather(w2v_table, idx, n, d)
    return out.reshape(b, l, d)


# speedup vs baseline: 1.8311x; 1.8311x over previous
import functools

import jax
import jax.numpy as jnp
from jax import lax
from jax.experimental import pallas as pl
from jax.experimental.pallas import tpu as pltpu
from jax.experimental.pallas import tpu_sc as plsc


@functools.partial(jax.jit, static_argnames=("n", "d"))
def _sc_gather(table, idx, n, d):
    info = plsc.get_sparse_core_info()
    nc, ns = info.num_cores, info.num_subcores
    nw = nc * ns
    assert n % nw == 0
    b_per_w = n // nw

    chunk = b_per_w
    while chunk > 1024 or chunk % 8 != 0:
        chunk //= 2
    n_chunks = b_per_w // chunk

    mesh = plsc.VectorSubcoreMesh(core_axis_name="c", subcore_axis_name="s")

    @functools.partial(
        pl.kernel,
        mesh=mesh,
        out_type=jax.ShapeDtypeStruct((n, d), jnp.float32),
        compiler_params=pltpu.CompilerParams(use_tc_tiling_on_sc=False),
        scratch_types=[
            pltpu.VMEM((chunk,), jnp.int32),
            pltpu.VMEM((chunk, d), jnp.float32),
            pltpu.SemaphoreType.DMA,
        ],
    )
    def k(table_hbm, idx_hbm, out_hbm, idx_v, rows_v, sem):
        wid = lax.axis_index("s") * nc + lax.axis_index("c")
        base = wid * b_per_w

        def step(g, carry):
            off = base + g * chunk
            pltpu.sync_copy(idx_hbm.at[pl.ds(off, chunk)], idx_v)
            pltpu.async_copy(table_hbm.at[idx_v], rows_v, sem).wait()
            pltpu.sync_copy(rows_v, out_hbm.at[pl.ds(off, chunk)])
            return carry

        lax.fori_loop(0, n_chunks, step, 0)

    return k(table, idx)


def kernel(text_vec, w2v_table):
    b, l = text_vec.shape
    v, d = w2v_table.shape
    n = b * l
    idx = text_vec.reshape(n).astype(jnp.int32)
    out = _sc_gather(w2v_table, idx, n, d)
    return out.reshape(b, l, d)


# speedup vs baseline: 1.8638x; 1.0179x over previous
import functools

import jax
import jax.numpy as jnp
from jax import lax
from jax.experimental import pallas as pl
from jax.experimental.pallas import tpu as pltpu
from jax.experimental.pallas import tpu_sc as plsc


@functools.partial(jax.jit, static_argnames=("n", "d"))
def _sc_gather(table, idx, n, d):
    info = plsc.get_sparse_core_info()
    nc, ns = info.num_cores, info.num_subcores
    nw = nc * ns
    assert n % nw == 0
    b_per_w = n // nw

    chunk = b_per_w
    while chunk > 1024 or chunk % 8 != 0:
        chunk //= 2
    n_chunks = b_per_w // chunk
    assert n_chunks % 2 == 0 and n_chunks >= 4

    mesh = plsc.VectorSubcoreMesh(core_axis_name="c", subcore_axis_name="s")

    @functools.partial(
        pl.kernel,
        mesh=mesh,
        out_type=jax.ShapeDtypeStruct((n, d), jnp.float32),
        compiler_params=pltpu.CompilerParams(use_tc_tiling_on_sc=False),
        scratch_types=[
            pltpu.VMEM((chunk,), jnp.int32),
            pltpu.VMEM((chunk,), jnp.int32),
            pltpu.VMEM((chunk, d), jnp.float32),
            pltpu.VMEM((chunk, d), jnp.float32),
            pltpu.SemaphoreType.DMA,
            pltpu.SemaphoreType.DMA,
            pltpu.SemaphoreType.DMA,
            pltpu.SemaphoreType.DMA,
        ],
    )
    def k(table_hbm, idx_hbm, out_hbm, idx0, idx1, rows0, rows1,
          g0, g1, w0, w1):
        wid = lax.axis_index("s") * nc + lax.axis_index("c")
        base = wid * b_per_w
        idx_v = (idx0, idx1)
        rows_v = (rows0, rows1)
        gsem = (g0, g1)
        wsem = (w0, w1)

        def start_gather(g, b):
            off = base + g * chunk
            pltpu.sync_copy(idx_hbm.at[pl.ds(off, chunk)], idx_v[b])
            pltpu.async_copy(table_hbm.at[idx_v[b]], rows_v[b], gsem[b])

        def wait_gather(b):
            pltpu.make_async_copy(
                table_hbm.at[idx_v[b]], rows_v[b], gsem[b]
            ).wait()

        def start_wb(g, b):
            off = base + g * chunk
            pltpu.async_copy(rows_v[b], out_hbm.at[pl.ds(off, chunk)], wsem[b])

        def wait_wb(g, b):
            off = base + g * chunk
            pltpu.make_async_copy(
                rows_v[b], out_hbm.at[pl.ds(off, chunk)], wsem[b]
            ).wait()

        start_gather(0, 0)

        def step(i, carry):
            go = 2 * i + 1
            wait_gather(0)
            start_wb(go - 1, 0)
            @pl.when(i > 0)
            def _():
                wait_wb(go - 2, 1)
            start_gather(go, 1)
            ge = 2 * i + 2
            wait_gather(1)
            start_wb(ge - 1, 1)
            wait_wb(ge - 2, 0)
            start_gather(ge, 0)
            return carry

        lax.fori_loop(0, (n_chunks - 2) // 2, step, 0)

        last = n_chunks - 1
        wait_gather(0)
        start_wb(last - 1, 0)
        wait_wb(last - 2, 1)
        start_gather(last, 1)
        wait_gather(1)
        wait_wb(last - 1, 0)
        start_wb(last, 1)
        wait_wb(last, 1)

    return k(table, idx)


def kernel(text_vec, w2v_table):
    b, l = text_vec.shape
    v, d = w2v_table.shape
    n = b * l
    idx = text_vec.reshape(n).astype(jnp.int32)
    out = _sc_gather(w2v_table, idx, n, d)
    return out.reshape(b, l, d)


# speedup vs baseline: 1.8748x; 1.0059x over previous
import functools

import jax
import jax.numpy as jnp
from jax import lax
from jax.experimental import pallas as pl
from jax.experimental.pallas import tpu as pltpu
from jax.experimental.pallas import tpu_sc as plsc

_NB = 4


@functools.partial(jax.jit, static_argnames=("n", "d"))
def _sc_gather(table, idx, n, d):
    info = plsc.get_sparse_core_info()
    nc, ns = info.num_cores, info.num_subcores
    nw = nc * ns
    assert n % nw == 0
    b_per_w = n // nw

    chunk = b_per_w
    while b_per_w + _NB * chunk * d > 128000 or chunk % 8 != 0:
        chunk //= 2
    n_chunks = b_per_w // chunk
    assert n_chunks % _NB == 0 and n_chunks >= 2 * _NB

    mesh = plsc.VectorSubcoreMesh(core_axis_name="c", subcore_axis_name="s")

    @functools.partial(
        pl.kernel,
        mesh=mesh,
        out_type=jax.ShapeDtypeStruct((n, d), jnp.float32),
        compiler_params=pltpu.CompilerParams(use_tc_tiling_on_sc=False),
        scratch_types=[
            pltpu.VMEM((b_per_w,), jnp.int32),
        ]
        + [pltpu.VMEM((chunk, d), jnp.float32) for _ in range(_NB)]
        + [pltpu.SemaphoreType.DMA for _ in range(2 * _NB)],
    )
    def k(table_hbm, idx_hbm, out_hbm, idx_all, *bufs_and_sems):
        rows_v = bufs_and_sems[:_NB]
        gsem = bufs_and_sems[_NB:2 * _NB]
        wsem = bufs_and_sems[2 * _NB:]
        wid = lax.axis_index("s") * nc + lax.axis_index("c")
        base = wid * b_per_w

        pltpu.sync_copy(idx_hbm.at[pl.ds(base, b_per_w)], idx_all)

        def start_gather(g, b):
            pltpu.async_copy(
                table_hbm.at[idx_all.at[pl.ds(g * chunk, chunk)]],
                rows_v[b], gsem[b])

        def wait_gather(g, b):
            pltpu.make_async_copy(
                table_hbm.at[idx_all.at[pl.ds(g * chunk, chunk)]],
                rows_v[b], gsem[b]).wait()

        def start_wb(g, b):
            pltpu.async_copy(
                rows_v[b], out_hbm.at[pl.ds(base + g * chunk, chunk)], wsem[b])

        def wait_wb(g, b):
            pltpu.make_async_copy(
                rows_v[b], out_hbm.at[pl.ds(base + g * chunk, chunk)],
                wsem[b]).wait()

        def substep(g, b):
            wait_wb(g - _NB, b)
            start_gather(g, b)
            bp = (_NB + b - (_NB - 1)) % _NB
            wait_gather(g - (_NB - 1), bp)
            start_wb(g - (_NB - 1), bp)

        for g in range(_NB - 1):
            start_gather(g, g)
        start_gather(_NB - 1, _NB - 1)
        wait_gather(0, 0)
        start_wb(0, 0)

        def step(i, carry):
            g0 = _NB * i
            for r in range(_NB):
                substep(g0 + r, r)
            return carry

        lax.fori_loop(1, n_chunks // _NB, step, 0)

        last = n_chunks - 1
        for r in range(_NB - 1):
            g = last - (_NB - 2) + r
            b = g % _NB
            wait_gather(g, b)
            start_wb(g, b)
        for r in range(_NB):
            g = last - (_NB - 1) + r
            wait_wb(g, g % _NB)

    return k(table, idx)


def kernel(text_vec, w2v_table):
    b, l = text_vec.shape
    v, d = w2v_table.shape
    n = b * l
    idx = text_vec.reshape(n).astype(jnp.int32)
    out = _sc_gather(w2v_table, idx, n, d)
    return out.reshape(b, l, d)
